# baseline (speedup 1.0000x reference)
import jax
import jax.numpy as jnp
from jax.experimental import pallas as pl
from jax.experimental.pallas import tpu as pltpu

_BM = 512


def _make_body(B, N, C):
    def body(m_ref, xr_ref, o_ref):
        res_t = jax.lax.dot_general(
            xr_ref[...],
            m_ref[...],
            dimension_numbers=(((1,), (1,)), ((), ())),
            preferred_element_type=jnp.float32,
            precision=jax.lax.Precision.DEFAULT,
        )
        o_ref[...] = res_t.reshape(B, C, m_ref.shape[0])

    return body


def kernel(M, x):
    B, N, C = x.shape
    xr = x.transpose(0, 2, 1).reshape(B * C, N)
    out_t = pl.pallas_call(
        _make_body(B, N, C),
        grid=(N // _BM,),
        in_specs=[
            pl.BlockSpec((_BM, N), lambda i: (i, 0)),
            pl.BlockSpec((B * C, N), lambda i: (0, 0)),
        ],
        out_specs=pl.BlockSpec((B, C, _BM), lambda i: (0, 0, i)),
        out_shape=jax.ShapeDtypeStruct((B, C, N), jnp.float32),
    )(M, xr)
    return out_t.transpose(0, 2, 1)


# speedup vs baseline: 1.0168x; 1.0168x over previous
import jax
import jax.numpy as jnp
from jax.experimental import pallas as pl

_BM = 1024


def _make_body(B, N, C):
    def body(m_ref, xr_ref, o_ref):
        res_t = jax.lax.dot_general(
            xr_ref[...],
            m_ref[...],
            dimension_numbers=(((1,), (1,)), ((), ())),
            preferred_element_type=jnp.float32,
            precision=jax.lax.Precision.DEFAULT,
        )
        o_ref[...] = res_t.reshape(B, C, m_ref.shape[0])

    return body


def kernel(M, x):
    B, N, C = x.shape
    xr = x.transpose(0, 2, 1).reshape(B * C, N)
    out_t = pl.pallas_call(
        _make_body(B, N, C),
        grid=(N // _BM,),
        in_specs=[
            pl.BlockSpec((_BM, N), lambda i: (i, 0)),
            pl.BlockSpec((B * C, N), lambda i: (0, 0)),
        ],
        out_specs=pl.BlockSpec((B, C, _BM), lambda i: (0, 0, i)),
        out_shape=jax.ShapeDtypeStruct((B, C, N), jnp.float32),
    )(M, xr)
    return out_t.transpose(0, 2, 1)
